# baseline (speedup 1.0000x reference)
import functools

import jax
import jax.numpy as jnp
from jax import lax
from jax.experimental import pallas as pl
from jax.experimental.pallas import tpu as pltpu
from jax.experimental.pallas import tpu_sc as plsc

N_EMB = 8192
DIM = 32
N_ROWS = 4096
BM = 256
GRID = N_ROWS // BM

_SC_INFO = plsc.get_sparse_core_info()
NC = _SC_INFO.num_cores
NS = _SC_INFO.num_subcores
NW = NC * NS
BPW = N_ROWS // NW
HPW = N_EMB // NS


def _argmin_body(z_ref, wt_ref, idx_ref):
    z = z_ref[...]
    wt = wt_ref[...]
    zz = jnp.sum(z * z, axis=1, keepdims=True)
    wsq = jnp.sum(wt * wt, axis=0, keepdims=True)
    m = jax.lax.dot_general(z, wt, (((1,), (0,)), ((), ())),
                            preferred_element_type=jnp.float32)
    d = (zz - 2.0 * m) + wsq
    vmin = jnp.min(d, axis=1, keepdims=True)
    cols = lax.broadcasted_iota(jnp.int32, (BM, N_EMB), 1)
    idx = jnp.min(jnp.where(d == vmin, cols, jnp.int32(2**30)), axis=1)
    idx_ref[...] = idx.reshape(1, 1, BM)


def _argmin_call(z_flat, wt):
    return pl.pallas_call(
        _argmin_body,
        grid=(GRID,),
        in_specs=[
            pl.BlockSpec((BM, DIM), lambda i: (i, 0)),
            pl.BlockSpec((DIM, N_EMB), lambda i: (0, 0)),
        ],
        out_specs=pl.BlockSpec((1, 1, BM), lambda i: (i, 0, 0)),
        out_shape=jax.ShapeDtypeStruct((GRID, 1, BM), jnp.int32),
    )(z_flat, wt)


def _sc_body(w_hbm, idx_hbm, zq_hbm, counts_hbm,
             idx_v, rows_v, ones_v, zeros_v, hist_sh, sem):
    c = lax.axis_index("c")
    s = lax.axis_index("s")
    wid = s * NC + c
    base = wid * BPW

    pltpu.sync_copy(idx_hbm.at[pl.ds(base, BPW)], idx_v)
    pltpu.async_copy(w_hbm.at[idx_v], rows_v, sem).wait()
    pltpu.sync_copy(rows_v, zq_hbm.at[pl.ds(base, BPW)])

    def _fill_zeros(i, _):
        zeros_v[pl.ds(i * 16, 16)] = jnp.zeros((16,), jnp.float32)
        return _
    lax.fori_loop(0, HPW // 16, _fill_zeros, None)

    def _fill_ones(i, _):
        ones_v[pl.ds(i * 16, 16)] = jnp.full((16,), 1.0, jnp.float32)
        return _
    lax.fori_loop(0, BPW // 16, _fill_ones, None)
    pltpu.sync_copy(zeros_v, hist_sh.at[pl.ds(s * HPW, HPW)])
    plsc.subcore_barrier()
    pltpu.sync_copy(ones_v, hist_sh.at[idx_v], add=True)
    plsc.subcore_barrier()

    @pl.when(s == 0)
    def _():
        pltpu.sync_copy(hist_sh, counts_hbm.at[c])


def _sc_call(W, idx):
    mesh = plsc.VectorSubcoreMesh(core_axis_name="c", subcore_axis_name="s")
    f = pl.kernel(
        _sc_body,
        out_type=[
            jax.ShapeDtypeStruct((N_ROWS, DIM), jnp.float32),
            jax.ShapeDtypeStruct((NC, N_EMB), jnp.float32),
        ],
        mesh=mesh,
        scratch_types=[
            pltpu.VMEM((BPW,), jnp.int32),
            pltpu.VMEM((BPW, DIM), jnp.float32),
            pltpu.VMEM((BPW,), jnp.float32),
            pltpu.VMEM((HPW,), jnp.float32),
            pltpu.VMEM_SHARED((N_EMB,), jnp.float32),
            pltpu.SemaphoreType.DMA,
        ],
        compiler_params=pltpu.CompilerParams(use_tc_tiling_on_sc=False),
    )
    return f(W, idx)


def _loss_body(z_ref, q_ref, c_ref, qst_ref, loss_ref, perp_ref):
    z = z_ref[...]
    q = q_ref[...]
    qst_ref[...] = z + (q - z)
    diff = z - q
    msq = jnp.mean(diff * diff)
    loss_ref[...] = (0.25 * msq + msq).reshape(1, 1)
    cc = c_ref[0:1, :] + c_ref[1:2, :]
    p = cc * (1.0 / N_ROWS)
    ent = p * jnp.log(p + 1e-10)
    perp_ref[...] = jnp.exp(-jnp.sum(ent)).reshape(1, 1)


def _loss_call(z_flat, zq_flat, counts2):
    return pl.pallas_call(
        _loss_body,
        out_shape=[
            jax.ShapeDtypeStruct((N_ROWS, DIM), jnp.float32),
            jax.ShapeDtypeStruct((1, 1), jnp.float32),
            jax.ShapeDtypeStruct((1, 1), jnp.float32),
        ],
    )(z_flat, zq_flat, counts2)


def kernel(z_e, W):
    b, c, h, w = z_e.shape
    z_flat = jnp.transpose(z_e, (0, 2, 3, 1)).reshape(-1, c)
    wt = W.T
    encoding_indices = _argmin_call(z_flat, wt).reshape(-1)
    zq_flat, counts2 = _sc_call(W, encoding_indices)
    zqst_flat, vq_loss, perplexity = _loss_call(z_flat, zq_flat, counts2)
    z_q_st = jnp.transpose(zqst_flat.reshape(b, h, w, c), (0, 3, 1, 2))
    indices = encoding_indices.reshape(b, h, w)
    return (z_q_st, vq_loss[0, 0], perplexity[0, 0], indices)


# speedup vs baseline: 1.0401x; 1.0401x over previous
import functools

import jax
import jax.numpy as jnp
from jax import lax
from jax.experimental import pallas as pl
from jax.experimental.pallas import tpu as pltpu
from jax.experimental.pallas import tpu_sc as plsc

N_EMB = 8192
DIM = 32
N_ROWS = 4096
BM = 256
GRID = N_ROWS // BM

NC = 2
NS = 16
NW = NC * NS
BPW = N_ROWS // NW
HPW = N_EMB // NS


def _argmin_body(z_ref, wt_ref, idx_ref):
    z = z_ref[...]
    wt = wt_ref[...]
    zz = jnp.sum(z * z, axis=1, keepdims=True)
    wsq = jnp.sum(wt * wt, axis=0, keepdims=True)
    m2 = jax.lax.dot_general(z + z, wt, (((1,), (0,)), ((), ())),
                             preferred_element_type=jnp.float32)
    half = N_EMB // 2
    a = (zz - m2[:, :half]) + wsq[:, :half]
    b = (zz - m2[:, half:]) + wsq[:, half:]
    i = jnp.where(b < a, jnp.int32(half), jnp.int32(0))
    v = jnp.minimum(a, b)
    half //= 2
    while half >= 128:
        a = v[:, :half]
        b = v[:, half:]
        ia = i[:, :half]
        ib = i[:, half:]
        i = jnp.where(b < a, ib + jnp.int32(half), ia)
        v = jnp.minimum(a, b)
        half //= 2
    vmin = jnp.min(v, axis=1, keepdims=True)
    lanes = lax.broadcasted_iota(jnp.int32, (BM, 128), 1)
    idx = jnp.min(jnp.where(v == vmin, i + lanes, jnp.int32(2**30)), axis=1)
    idx_ref[...] = idx.reshape(1, 1, BM)


def _argmin_call(z_flat, wt):
    return pl.pallas_call(
        _argmin_body,
        grid=(GRID,),
        in_specs=[
            pl.BlockSpec((BM, DIM), lambda i: (i, 0)),
            pl.BlockSpec((DIM, N_EMB), lambda i: (0, 0)),
        ],
        out_specs=pl.BlockSpec((1, 1, BM), lambda i: (i, 0, 0)),
        out_shape=jax.ShapeDtypeStruct((GRID, 1, BM), jnp.int32),
    )(z_flat, wt)


def _sc_body(w_hbm, idx_hbm, zq_hbm, counts_hbm,
             idx_v, rows_v, ones_v, zeros_v, hist_sh, sem):
    c = lax.axis_index("c")
    s = lax.axis_index("s")
    wid = s * NC + c
    base = wid * BPW

    pltpu.sync_copy(idx_hbm.at[pl.ds(base, BPW)], idx_v)
    pltpu.async_copy(w_hbm.at[idx_v], rows_v, sem).wait()
    pltpu.sync_copy(rows_v, zq_hbm.at[pl.ds(base, BPW)])

    def _fill_zeros(i, _):
        zeros_v[pl.ds(i * 16, 16)] = jnp.zeros((16,), jnp.float32)
        return _
    lax.fori_loop(0, HPW // 16, _fill_zeros, None)

    def _fill_ones(i, _):
        ones_v[pl.ds(i * 16, 16)] = jnp.full((16,), 1.0, jnp.float32)
        return _
    lax.fori_loop(0, BPW // 16, _fill_ones, None)
    pltpu.sync_copy(zeros_v, hist_sh.at[pl.ds(s * HPW, HPW)])
    plsc.subcore_barrier()
    pltpu.sync_copy(ones_v, hist_sh.at[idx_v], add=True)
    plsc.subcore_barrier()

    @pl.when(s == 0)
    def _():
        pltpu.sync_copy(hist_sh, counts_hbm.at[c])


def _sc_call(W, idx):
    mesh = plsc.VectorSubcoreMesh(core_axis_name="c", subcore_axis_name="s")
    f = pl.kernel(
        _sc_body,
        out_type=[
            jax.ShapeDtypeStruct((N_ROWS, DIM), jnp.float32),
            jax.ShapeDtypeStruct((NC, N_EMB), jnp.float32),
        ],
        mesh=mesh,
        scratch_types=[
            pltpu.VMEM((BPW,), jnp.int32),
            pltpu.VMEM((BPW, DIM), jnp.float32),
            pltpu.VMEM((BPW,), jnp.float32),
            pltpu.VMEM((HPW,), jnp.float32),
            pltpu.VMEM_SHARED((N_EMB,), jnp.float32),
            pltpu.SemaphoreType.DMA,
        ],
        compiler_params=pltpu.CompilerParams(use_tc_tiling_on_sc=False),
    )
    return f(W, idx)


def _loss_body(z_ref, q_ref, c_ref, qst_ref, loss_ref, perp_ref):
    z = z_ref[...]
    q = q_ref[...]
    qst_ref[...] = z + (q - z)
    diff = z - q
    msq = jnp.mean(diff * diff)
    loss_ref[...] = (0.25 * msq + msq).reshape(1, 1)
    cc = c_ref[0:1, :] + c_ref[1:2, :]
    p = cc * (1.0 / N_ROWS)
    ent = p * jnp.log(p + 1e-10)
    perp_ref[...] = jnp.exp(-jnp.sum(ent)).reshape(1, 1)


def _loss_call(z_flat, zq_flat, counts2):
    return pl.pallas_call(
        _loss_body,
        out_shape=[
            jax.ShapeDtypeStruct((N_ROWS, DIM), jnp.float32),
            jax.ShapeDtypeStruct((1, 1), jnp.float32),
            jax.ShapeDtypeStruct((1, 1), jnp.float32),
        ],
    )(z_flat, zq_flat, counts2)


def kernel(z_e, W):
    b, c, h, w = z_e.shape
    z_flat = jnp.transpose(z_e, (0, 2, 3, 1)).reshape(-1, c)
    wt = W.T
    encoding_indices = _argmin_call(z_flat, wt).reshape(-1)
    zq_flat, counts2 = _sc_call(W, encoding_indices)
    zqst_flat, vq_loss, perplexity = _loss_call(z_flat, zq_flat, counts2)
    z_q_st = jnp.transpose(zqst_flat.reshape(b, h, w, c), (0, 3, 1, 2))
    indices = encoding_indices.reshape(b, h, w)
    return (z_q_st, vq_loss[0, 0], perplexity[0, 0], indices)


# speedup vs baseline: 1.0788x; 1.0372x over previous
import functools

import jax
import jax.numpy as jnp
from jax import lax
from jax.experimental import pallas as pl
from jax.experimental.pallas import tpu as pltpu
from jax.experimental.pallas import tpu_sc as plsc

N_EMB = 8192
DIM = 32
N_ROWS = 4096
BM = 256
GRID = N_ROWS // BM

NC = 2
NS = 16
NW = NC * NS
BPW = N_ROWS // NW
HPW = N_EMB // NS


def _argmin_body(z_ref, w_ref, idx_ref):
    zb = z_ref[0]
    w = w_ref[...]
    zz = jnp.sum(zb * zb, axis=0, keepdims=True)
    wsq = jnp.sum(w * w, axis=1, keepdims=True)
    m2 = jax.lax.dot_general(w + w, zb, (((1,), (0,)), ((), ())),
                             preferred_element_type=jnp.float32)
    def _split(x):
        n = x.shape[0]
        x3 = x.reshape(n // 16, 16, BM)
        return (x3[:, :8, :].reshape(n // 2, BM),
                x3[:, 8:, :].reshape(n // 2, BM))

    d = (zz - m2) + wsq
    a, b = _split(d)
    i = jnp.where(b < a, jnp.int32(8), jnp.int32(0))
    v = jnp.minimum(a, b)
    step = 16
    while v.shape[0] > 8:
        a, b = _split(v)
        ia, ib = _split(i)
        i = jnp.where(b < a, ib + jnp.int32(step), ia)
        v = jnp.minimum(a, b)
        step *= 2
    vmin = jnp.min(v, axis=0, keepdims=True)
    rows = lax.broadcasted_iota(jnp.int32, (8, BM), 0)
    idx = jnp.min(jnp.where(v == vmin, i + rows, jnp.int32(2**30)), axis=0)
    idx_ref[...] = idx.reshape(1, 1, BM)


def _argmin_call(z_nat, w):
    blocks_per_batch = 1024 // BM
    return pl.pallas_call(
        _argmin_body,
        grid=(GRID,),
        in_specs=[
            pl.BlockSpec((1, DIM, BM),
                         lambda i: (i // blocks_per_batch, 0,
                                    i % blocks_per_batch)),
            pl.BlockSpec((N_EMB, DIM), lambda i: (0, 0)),
        ],
        out_specs=pl.BlockSpec((1, 1, BM), lambda i: (i, 0, 0)),
        out_shape=jax.ShapeDtypeStruct((GRID, 1, BM), jnp.int32),
    )(z_nat, w)


def _sc_body(w_hbm, idx_hbm, zq_hbm, counts_hbm,
             idx_v, rows_v, ones_v, zeros_v, hist_sh, sem):
    c = lax.axis_index("c")
    s = lax.axis_index("s")
    wid = s * NC + c
    base = wid * BPW

    pltpu.sync_copy(idx_hbm.at[pl.ds(base, BPW)], idx_v)
    pltpu.async_copy(w_hbm.at[idx_v], rows_v, sem).wait()
    pltpu.sync_copy(rows_v, zq_hbm.at[pl.ds(base, BPW)])

    def _fill_zeros(i, _):
        zeros_v[pl.ds(i * 16, 16)] = jnp.zeros((16,), jnp.float32)
        return _
    lax.fori_loop(0, HPW // 16, _fill_zeros, None)

    def _fill_ones(i, _):
        ones_v[pl.ds(i * 16, 16)] = jnp.full((16,), 1.0, jnp.float32)
        return _
    lax.fori_loop(0, BPW // 16, _fill_ones, None)
    pltpu.sync_copy(zeros_v, hist_sh.at[pl.ds(s * HPW, HPW)])
    plsc.subcore_barrier()
    pltpu.sync_copy(ones_v, hist_sh.at[idx_v], add=True)
    plsc.subcore_barrier()

    @pl.when(s == 0)
    def _():
        pltpu.sync_copy(hist_sh, counts_hbm.at[c])


def _sc_call(W, idx):
    mesh = plsc.VectorSubcoreMesh(core_axis_name="c", subcore_axis_name="s")
    f = pl.kernel(
        _sc_body,
        out_type=[
            jax.ShapeDtypeStruct((N_ROWS, DIM), jnp.float32),
            jax.ShapeDtypeStruct((NC, N_EMB), jnp.float32),
        ],
        mesh=mesh,
        scratch_types=[
            pltpu.VMEM((BPW,), jnp.int32),
            pltpu.VMEM((BPW, DIM), jnp.float32),
            pltpu.VMEM((BPW,), jnp.float32),
            pltpu.VMEM((HPW,), jnp.float32),
            pltpu.VMEM_SHARED((N_EMB,), jnp.float32),
            pltpu.SemaphoreType.DMA,
        ],
        compiler_params=pltpu.CompilerParams(use_tc_tiling_on_sc=False),
    )
    return f(W, idx)


def _loss_body(z_ref, q_ref, c_ref, qst_ref, loss_ref, perp_ref):
    z = z_ref[...]
    q = q_ref[...]
    qst_ref[...] = z + (q - z)
    diff = z - q
    msq = jnp.mean(diff * diff)
    loss_ref[...] = (0.25 * msq + msq).reshape(1, 1)
    cc = c_ref[0:1, :] + c_ref[1:2, :]
    p = cc * (1.0 / N_ROWS)
    ent = p * jnp.log(p + 1e-10)
    perp_ref[...] = jnp.exp(-jnp.sum(ent)).reshape(1, 1)


def _loss_call(z_flat, zq_flat, counts2):
    return pl.pallas_call(
        _loss_body,
        out_shape=[
            jax.ShapeDtypeStruct((N_ROWS, DIM), jnp.float32),
            jax.ShapeDtypeStruct((1, 1), jnp.float32),
            jax.ShapeDtypeStruct((1, 1), jnp.float32),
        ],
    )(z_flat, zq_flat, counts2)


def kernel(z_e, W):
    b, c, h, w = z_e.shape
    z_nat = z_e.reshape(b, c, h * w)
    z_flat = jnp.transpose(z_e, (0, 2, 3, 1)).reshape(-1, c)
    encoding_indices = _argmin_call(z_nat, W).reshape(-1)
    zq_flat, counts2 = _sc_call(W, encoding_indices)
    zqst_flat, vq_loss, perplexity = _loss_call(z_flat, zq_flat, counts2)
    z_q_st = jnp.transpose(zqst_flat.reshape(b, h, w, c), (0, 3, 1, 2))
    indices = encoding_indices.reshape(b, h, w)
    return (z_q_st, vq_loss[0, 0], perplexity[0, 0], indices)
